# baseline (device time: 6399 ns/iter reference)
import jax
import jax.numpy as jnp
from jax import lax
from jax.experimental import pallas as pl
from jax.experimental.pallas import tpu as pltpu

N_CHUNKS = 4


def kernel(x, pi):
    rows = x.shape[1]
    assert rows % N_CHUNKS == 0
    chunk = rows // N_CHUNKS

    x = pltpu.with_memory_space_constraint(x, pltpu.MemorySpace.HBM)
    pi = pltpu.with_memory_space_constraint(pi, pltpu.MemorySpace.HBM)

    def body(
        pi_hbm,
        x_hbm,
        out_ref,
        x_vmem,
        comm_ref,
        pi_smem,
        copy_sems,
        send_sems,
        recv_sems,
    ):
        my_x = lax.axis_index("x")
        my_y = lax.axis_index("y")
        my_z = lax.axis_index("z")

        pi_cp = pltpu.make_async_copy(pi_hbm, pi_smem, copy_sems.at[0])
        pi_cp.start()
        x_cps = []
        for h in range(N_CHUNKS):
            sl = pl.ds(h * chunk, chunk)
            x_cp = pltpu.make_async_copy(
                x_hbm.at[0, sl, :], x_vmem.at[0, sl, :], copy_sems.at[1 + h]
            )
            x_cp.start()
            x_cps.append(x_cp)

        pi_cp.wait()
        dst_y = jnp.where(my_y == 0, pi_smem[0], pi_smem[1])
        is_swap = dst_y != my_y

        @pl.when(is_swap)
        def _():
            rdmas = []
            for h in range(N_CHUNKS):
                sl = pl.ds(h * chunk, chunk)
                x_cps[h].wait()
                comm_ref[0, sl, :] = x_vmem[0, sl, :].astype(jnp.bfloat16)
                rdma = pltpu.make_async_remote_copy(
                    src_ref=comm_ref.at[0, sl, :],
                    dst_ref=out_ref.at[0, sl, :],
                    send_sem=send_sems.at[h],
                    recv_sem=recv_sems.at[h],
                    device_id=(my_x, dst_y, my_z),
                    device_id_type=pl.DeviceIdType.MESH,
                )
                rdma.start()
                rdmas.append(rdma)
            for rdma in rdmas:
                rdma.wait()

        @pl.when(jnp.logical_not(is_swap))
        def _():
            for h in range(N_CHUNKS):
                x_cps[h].wait()
            out_ref[...] = x_vmem[...].astype(jnp.bfloat16)

    return pl.pallas_call(
        body,
        out_shape=jax.ShapeDtypeStruct(x.shape, jnp.bfloat16),
        in_specs=[
            pl.BlockSpec(memory_space=pltpu.MemorySpace.HBM),
            pl.BlockSpec(memory_space=pltpu.MemorySpace.HBM),
        ],
        out_specs=pl.BlockSpec(memory_space=pltpu.VMEM),
        scratch_shapes=[
            pltpu.VMEM(x.shape, x.dtype),
            pltpu.VMEM(x.shape, jnp.bfloat16),
            pltpu.SMEM((2,), jnp.int32),
            pltpu.SemaphoreType.DMA((1 + N_CHUNKS,)),
            pltpu.SemaphoreType.DMA((N_CHUNKS,)),
            pltpu.SemaphoreType.DMA((N_CHUNKS,)),
        ],
        compiler_params=pltpu.CompilerParams(
            skip_device_barrier=True,
        ),
    )(pi, x)


# device time: 6398 ns/iter; 1.0002x vs baseline; 1.0002x over previous
import jax
import jax.numpy as jnp
from jax import lax
from jax.experimental import pallas as pl
from jax.experimental.pallas import tpu as pltpu

N_CHUNKS = 2


def kernel(x, pi):
    rows = x.shape[1]
    assert rows % N_CHUNKS == 0
    chunk = rows // N_CHUNKS

    x = pltpu.with_memory_space_constraint(x, pltpu.MemorySpace.HBM)
    pi = pltpu.with_memory_space_constraint(pi, pltpu.MemorySpace.HBM)

    def body(
        pi_hbm,
        x_hbm,
        out_ref,
        x_vmem,
        comm_ref,
        pi_smem,
        copy_sems,
        send_sems,
        recv_sems,
    ):
        my_x = lax.axis_index("x")
        my_y = lax.axis_index("y")
        my_z = lax.axis_index("z")

        pi_cp = pltpu.make_async_copy(pi_hbm, pi_smem, copy_sems.at[0])
        pi_cp.start()
        x_cps = []
        for h in range(N_CHUNKS):
            sl = pl.ds(h * chunk, chunk)
            x_cp = pltpu.make_async_copy(
                x_hbm.at[0, sl, :], x_vmem.at[0, sl, :], copy_sems.at[1 + h]
            )
            x_cp.start()
            x_cps.append(x_cp)

        pi_cp.wait()
        dst_y = jnp.where(my_y == 0, pi_smem[0], pi_smem[1])
        is_swap = dst_y != my_y

        @pl.when(is_swap)
        def _():
            rdmas = []
            for h in range(N_CHUNKS):
                sl = pl.ds(h * chunk, chunk)
                x_cps[h].wait()
                comm_ref[0, sl, :] = x_vmem[0, sl, :].astype(jnp.bfloat16)
                rdma = pltpu.make_async_remote_copy(
                    src_ref=comm_ref.at[0, sl, :],
                    dst_ref=out_ref.at[0, sl, :],
                    send_sem=send_sems.at[h],
                    recv_sem=recv_sems.at[h],
                    device_id=(my_x, dst_y, my_z),
                    device_id_type=pl.DeviceIdType.MESH,
                )
                rdma.start()
                rdmas.append(rdma)
            for rdma in rdmas:
                rdma.wait()

        @pl.when(jnp.logical_not(is_swap))
        def _():
            for h in range(N_CHUNKS):
                x_cps[h].wait()
            out_ref[...] = x_vmem[...].astype(jnp.bfloat16)

    return pl.pallas_call(
        body,
        out_shape=jax.ShapeDtypeStruct(x.shape, jnp.bfloat16),
        in_specs=[
            pl.BlockSpec(memory_space=pltpu.MemorySpace.HBM),
            pl.BlockSpec(memory_space=pltpu.MemorySpace.HBM),
        ],
        out_specs=pl.BlockSpec(memory_space=pltpu.VMEM),
        scratch_shapes=[
            pltpu.VMEM(x.shape, x.dtype),
            pltpu.VMEM(x.shape, jnp.bfloat16),
            pltpu.SMEM((2,), jnp.int32),
            pltpu.SemaphoreType.DMA((1 + N_CHUNKS,)),
            pltpu.SemaphoreType.DMA((N_CHUNKS,)),
            pltpu.SemaphoreType.DMA((N_CHUNKS,)),
        ],
        compiler_params=pltpu.CompilerParams(
            skip_device_barrier=True,
        ),
    )(pi, x)
